# baseline (speedup 1.0000x reference)
import functools

import jax
import jax.numpy as jnp
from jax import lax
from jax.experimental import pallas as pl
from jax.experimental.pallas import tpu as pltpu
from jax.experimental.pallas import tpu_sc as plsc

_N = 10000
_E = 320000
_D = 128
_DH = _D // 2

_NC = 2
_NS = 16
_CHUNK = 80
_NCHUNK = 252
_NBUF = 4
_EPAD = _NS * _NCHUNK * _CHUNK
_GROUPS = _CHUNK // 16
_NPAD = 10240
_ROWS_PT = _NPAD // _NS
_ZROWS = 128
_LANES = _DH // 16


def _sc_aggregate(xs, src2, dst2, w2):
    mesh = plsc.VectorSubcoreMesh(core_axis_name="c", subcore_axis_name="s")

    @functools.partial(
        pl.kernel,
        mesh=mesh,
        out_type=jax.ShapeDtypeStruct((_NC, _NPAD, _DH), jnp.float32),
        compiler_params=pltpu.CompilerParams(use_tc_tiling_on_sc=False),
        scratch_types=[
            pltpu.VMEM((_NCHUNK, _CHUNK), jnp.int32),
            pltpu.VMEM((_NCHUNK, _CHUNK), jnp.int32),
            pltpu.VMEM((_NCHUNK, _CHUNK), jnp.float32),
            pltpu.VMEM((_CHUNK, _DH), jnp.float32),
            pltpu.VMEM((_CHUNK, _DH), jnp.float32),
            pltpu.VMEM((_CHUNK, _DH), jnp.float32),
            pltpu.VMEM((_CHUNK, _DH), jnp.float32),
            pltpu.VMEM((_ZROWS, _DH), jnp.float32),
            pltpu.VMEM_SHARED((_NPAD, _DH), jnp.float32),
            pltpu.SemaphoreType.DMA,
            pltpu.SemaphoreType.DMA,
            pltpu.SemaphoreType.DMA,
            pltpu.SemaphoreType.DMA,
            pltpu.SemaphoreType.DMA,
            pltpu.SemaphoreType.DMA,
            pltpu.SemaphoreType.DMA,
            pltpu.SemaphoreType.DMA,
        ],
    )
    def agg(xs_hbm, src_hbm, dst_hbm, w_hbm, out_hbm,
            src_v, dst_v, w_v, rows0, rows1, rows2, rows3, zrow_v, acc,
            gsem0, gsem1, gsem2, gsem3, ssem0, ssem1, ssem2, ssem3):
        rows_bufs = (rows0, rows1, rows2, rows3)
        gsems = (gsem0, gsem1, gsem2, gsem3)
        ssems = (ssem0, ssem1, ssem2, ssem3)
        c = lax.axis_index("c")
        s = lax.axis_index("s")

        zeros16 = jnp.zeros((16,), jnp.float32)

        def zero_row(r, carry):
            for l in range(_LANES):
                zrow_v[r, pl.ds(l * 16, 16)] = zeros16
            return carry

        lax.fori_loop(0, _ZROWS, zero_row, 0)

        base = s * _ROWS_PT
        for k in range(_ROWS_PT // _ZROWS):
            pltpu.sync_copy(zrow_v, acc.at[pl.ds(base + k * _ZROWS, _ZROWS)])

        plsc.subcore_barrier()

        pltpu.sync_copy(src_hbm.at[s], src_v)
        pltpu.sync_copy(dst_hbm.at[s], dst_v)
        pltpu.sync_copy(w_hbm.at[s], w_v)

        bcast_dn = lax.GatherDimensionNumbers(
            offset_dims=(), collapsed_slice_dims=(0,), start_index_map=(0,))

        def scale_rows(ci, rows):
            for g in range(_GROUPS):
                w16 = w_v[ci, pl.ds(g * 16, 16)]
                for i in range(16):
                    ws = lax.gather(
                        w16, jnp.full((16, 1), i, jnp.int32), bcast_dn, (1,),
                        mode=lax.GatherScatterMode.PROMISE_IN_BOUNDS)
                    e = g * 16 + i
                    for r in range(_LANES):
                        sl = pl.ds(r * 16, 16)
                        rows[e, sl] = rows[e, sl] * ws

        def gather(ci, rows, sem):
            return pltpu.async_copy(xs_hbm.at[c].at[src_v.at[ci]], rows, sem)

        def gather_wait(ci, rows, sem):
            pltpu.make_async_copy(xs_hbm.at[c].at[src_v.at[ci]], rows, sem).wait()

        def scatter(ci, rows, sem):
            return pltpu.async_copy(rows, acc.at[dst_v.at[ci]], sem, add=True)

        def scatter_wait(ci, rows, sem):
            pltpu.make_async_copy(rows, acc.at[dst_v.at[ci]], sem).wait()

        gather(0, rows0, gsem0)
        gather(1, rows1, gsem1)

        def ring(g, carry):
            for b in range(_NBUF):
                ci = _NBUF * g + b
                rb, gs, ss = rows_bufs[b], gsems[b], ssems[b]
                b2 = (b + 2) % _NBUF
                gather_wait(ci, rb, gs)

                @pl.when(ci >= 2)
                def _():
                    scatter_wait(ci - 2, rows_bufs[b2], ssems[b2])

                @pl.when(ci + 2 < _NCHUNK)
                def _():
                    gather(ci + 2, rows_bufs[b2], gsems[b2])

                scale_rows(ci, rb)
                scatter(ci, rb, ss)
            return carry

        lax.fori_loop(0, _NCHUNK // _NBUF, ring, 0)
        scatter_wait(_NCHUNK - 2, rows2, ssem2)
        scatter_wait(_NCHUNK - 1, rows3, ssem3)

        plsc.subcore_barrier()

        pltpu.sync_copy(acc.at[pl.ds(base, _ROWS_PT)],
                        out_hbm.at[c, pl.ds(base, _ROWS_PT)])

    return agg(xs, src2, dst2, w2)


def _tc_finish(partials, W):
    blk = 1000

    def body(p_ref, w_ref, o_ref):
        w = w_ref[...]
        o_ref[...] = jnp.maximum(
            jnp.dot(p_ref[0], w[:_DH, :], preferred_element_type=jnp.float32)
            + jnp.dot(p_ref[1], w[_DH:, :], preferred_element_type=jnp.float32),
            0.0)

    return pl.pallas_call(
        body,
        grid=(_N // blk,),
        in_specs=[
            pl.BlockSpec((_NC, blk, _DH), lambda i: (0, i, 0)),
            pl.BlockSpec((_D, _D), lambda i: (0, 0)),
        ],
        out_specs=pl.BlockSpec((blk, _D), lambda i: (i, 0)),
        out_shape=jax.ShapeDtypeStruct((_N, _D), jnp.float32),
    )(partials, W)


def kernel(x, edge_index, edge_weight, W):
    xs = jnp.stack([x[:, :_DH], x[:, _DH:]])
    pad = _EPAD - _E
    src2 = jnp.concatenate(
        [edge_index[0], jnp.zeros((pad,), jnp.int32)]).reshape(
            _NS, _NCHUNK, _CHUNK)
    dst2 = jnp.concatenate(
        [edge_index[1], jnp.zeros((pad,), jnp.int32)]).reshape(
            _NS, _NCHUNK, _CHUNK)
    w2 = jnp.concatenate(
        [edge_weight, jnp.zeros((pad,), jnp.float32)]).reshape(
            _NS, _NCHUNK, _CHUNK)
    partials = _sc_aggregate(xs, src2, dst2, w2)
    return _tc_finish(partials, W)


# speedup vs baseline: 1.0831x; 1.0831x over previous
import functools

import jax
import jax.numpy as jnp
from jax import lax
from jax.experimental import pallas as pl
from jax.experimental.pallas import tpu as pltpu
from jax.experimental.pallas import tpu_sc as plsc

_N = 10000
_E = 320000
_D = 128
_DH = _D // 2

_NC = 2
_NS = 16
_EPS = _E // _NS
_CHUNK = 80
_NCHUNK = _EPS // _CHUNK
_GROUPS = _CHUNK // 16
_NPAD = 10240
_ROWS_PT = _NPAD // _NS
_ZROWS = 128
_LANES = _DH // 16


def _sc_aggregate(xs, src2, dst2, w2):
    mesh = plsc.VectorSubcoreMesh(core_axis_name="c", subcore_axis_name="s")

    @functools.partial(
        pl.kernel,
        mesh=mesh,
        out_type=jax.ShapeDtypeStruct((_NC, _NPAD, _DH), jnp.float32),
        compiler_params=pltpu.CompilerParams(use_tc_tiling_on_sc=False),
        scratch_types=[
            pltpu.VMEM((_NCHUNK, _CHUNK), jnp.int32),
            pltpu.VMEM((_NCHUNK, _CHUNK), jnp.int32),
            pltpu.VMEM((_NCHUNK, _CHUNK), jnp.float32),
            pltpu.VMEM((_CHUNK, _DH), jnp.float32),
            pltpu.VMEM((_CHUNK, _DH), jnp.float32),
            pltpu.VMEM((_ZROWS, _DH), jnp.float32),
            pltpu.VMEM_SHARED((_NPAD, _DH), jnp.float32),
            pltpu.SemaphoreType.DMA,
            pltpu.SemaphoreType.DMA,
        ],
    )
    def agg(xs_hbm, src_hbm, dst_hbm, w_hbm, out_hbm,
            src_v, dst_v, w_v, rows0, rows1, zrow_v, acc, sem0, sem1):
        c = lax.axis_index("c")
        s = lax.axis_index("s")

        zeros16 = jnp.zeros((16,), jnp.float32)

        def zero_row(r, carry):
            for l in range(_LANES):
                zrow_v[r, pl.ds(l * 16, 16)] = zeros16
            return carry

        lax.fori_loop(0, _ZROWS, zero_row, 0)

        base = s * _ROWS_PT
        for k in range(_ROWS_PT // _ZROWS):
            pltpu.sync_copy(zrow_v, acc.at[pl.ds(base + k * _ZROWS, _ZROWS)])

        plsc.subcore_barrier()

        pltpu.sync_copy(src_hbm.at[s], src_v)
        pltpu.sync_copy(dst_hbm.at[s], dst_v)
        pltpu.sync_copy(w_hbm.at[s], w_v)

        bcast_dn = lax.GatherDimensionNumbers(
            offset_dims=(), collapsed_slice_dims=(0,), start_index_map=(0,))

        def scale_rows(ci, rows):
            for g in range(_GROUPS):
                w16 = w_v[ci, pl.ds(g * 16, 16)]
                for i in range(16):
                    ws = lax.gather(
                        w16, jnp.full((16, 1), i, jnp.int32), bcast_dn, (1,),
                        mode=lax.GatherScatterMode.PROMISE_IN_BOUNDS)
                    e = g * 16 + i
                    for r in range(_LANES):
                        sl = pl.ds(r * 16, 16)
                        rows[e, sl] = rows[e, sl] * ws

        def gather(ci, rows, sem):
            return pltpu.async_copy(xs_hbm.at[c].at[src_v.at[ci]], rows, sem)

        def gather_wait(ci, rows, sem):
            pltpu.make_async_copy(xs_hbm.at[c].at[src_v.at[ci]], rows, sem).wait()

        def commit(ci, rows):
            scale_rows(ci, rows)
            pltpu.sync_copy(rows, acc.at[dst_v.at[ci]], add=True)

        gather(0, rows0, sem0)

        def pipe(o, carry):
            ca = 2 * o
            cb = 2 * o + 1
            gather(cb, rows1, sem1)
            gather_wait(ca, rows0, sem0)
            commit(ca, rows0)

            @pl.when(cb + 1 < _NCHUNK)
            def _():
                gather(cb + 1, rows0, sem0)

            gather_wait(cb, rows1, sem1)
            commit(cb, rows1)
            return carry

        lax.fori_loop(0, _NCHUNK // 2, pipe, 0)

        plsc.subcore_barrier()

        pltpu.sync_copy(acc.at[pl.ds(base, _ROWS_PT)],
                        out_hbm.at[c, pl.ds(base, _ROWS_PT)])

    return agg(xs, src2, dst2, w2)


def _tc_finish(partials, W):
    blk = 1000

    def body(p_ref, w_ref, o_ref):
        w = w_ref[...]
        o_ref[...] = jnp.maximum(
            jnp.dot(p_ref[0], w[:_DH, :], preferred_element_type=jnp.float32)
            + jnp.dot(p_ref[1], w[_DH:, :], preferred_element_type=jnp.float32),
            0.0)

    return pl.pallas_call(
        body,
        grid=(_N // blk,),
        in_specs=[
            pl.BlockSpec((_NC, blk, _DH), lambda i: (0, i, 0)),
            pl.BlockSpec((_D, _D), lambda i: (0, 0)),
        ],
        out_specs=pl.BlockSpec((blk, _D), lambda i: (i, 0)),
        out_shape=jax.ShapeDtypeStruct((_N, _D), jnp.float32),
    )(partials, W)


def kernel(x, edge_index, edge_weight, W):
    xs = jnp.stack([x[:, :_DH], x[:, _DH:]])
    src2 = edge_index[0].reshape(_NS, _NCHUNK, _CHUNK)
    dst2 = edge_index[1].reshape(_NS, _NCHUNK, _CHUNK)
    w2 = edge_weight.reshape(_NS, _NCHUNK, _CHUNK)
    partials = _sc_aggregate(xs, src2, dst2, w2)
    return _tc_finish(partials, W)
